# Initial kernel scaffold; baseline (speedup 1.0000x reference)
import jax
import jax.numpy as jnp
from jax.experimental import pallas as pl


def kernel(x_e, x_r, edge_index, edge_type, w_loop, w_fwd, w_bwd, w_rel, self_loop, bias, bn_gamma, bn_beta):
    raise NotImplementedError("write your pallas kernel here")


# speedup vs baseline: 5.7034x; 5.7034x over previous
import functools

import jax
import jax.numpy as jnp
from jax import lax
from jax.experimental import pallas as pl
from jax.experimental.pallas import tpu as pltpu
from jax.experimental.pallas import tpu_sc as plsc

N = 10000
E = 320000
D = 128
NREL = 100
CW = 128

NP = 10240
NSUB = 16
K = 128
EPT = 20480
EPAD = NSUB * EPT
NCH = EPT // K
RPT = NP // NSUB

_MESH = plsc.VectorSubcoreMesh(core_axis_name="c", subcore_axis_name="s")
_SC_PARAMS = pltpu.CompilerParams(needs_layout_passes=False)


def _f32(shape):
    return jax.ShapeDtypeStruct(shape, jnp.float32)


@functools.partial(
    pl.kernel,
    out_type=(_f32((NP,)), _f32((NP,))),
    mesh=_MESH,
    scratch_types=[
        pltpu.VMEM((K,), jnp.int32),
        pltpu.VMEM((K,), jnp.float32),
        pltpu.VMEM((RPT,), jnp.float32),
        pltpu.VMEM_SHARED((NP,), jnp.float32),
    ],
    compiler_params=_SC_PARAMS,
)
def _sc_degrees(src_hbm, tgt_hbm, degs_out, degt_out, idx_v, ones_v, row_v,
                acc_sh):
    s = lax.axis_index("s")
    c = lax.axis_index("c")

    def fill(ref, n, value):
        def body(i, _):
            ref[pl.ds(i * 16, 16)] = jnp.full((16,), value, jnp.float32)
            return 0
        lax.fori_loop(0, n // 16, body, 0)

    def run(eidx_hbm, out_hbm, acc_sh):
        fill(ones_v, K, 1.0)
        fill(row_v, RPT, 0.0)
        pltpu.sync_copy(row_v, acc_sh.at[pl.ds(s * RPT, RPT)])
        plsc.subcore_barrier()
        base = s * EPT

        def step(i, _):
            off = pl.multiple_of(base + i * K, K)
            pltpu.sync_copy(eidx_hbm.at[pl.ds(off, K)], idx_v)
            pltpu.sync_copy(ones_v, acc_sh.at[idx_v], add=True)
            return 0

        lax.fori_loop(0, NCH, step, 0)
        plsc.subcore_barrier()
        pltpu.sync_copy(acc_sh.at[pl.ds(s * RPT, RPT)], row_v)
        pltpu.sync_copy(row_v, out_hbm.at[pl.ds(s * RPT, RPT)])

    @pl.when(c == 0)
    def _():
        run(src_hbm, degs_out, acc_sh)

    @pl.when(c == 1)
    def _():
        run(tgt_hbm, degt_out, acc_sh)


_CPT = NP * CW // NSUB
_ZC = 8192

@functools.partial(
    pl.kernel,
    out_type=(_f32((NP * CW,)), _f32((NP * CW,))),
    mesh=_MESH,
    scratch_types=[
        pltpu.VMEM((K,), jnp.int32),
        pltpu.VMEM((K,), jnp.int32),
        pltpu.VMEM((K,), jnp.int32),
        pltpu.VMEM((K,), jnp.int32),
        pltpu.VMEM((K,), jnp.float32),
        pltpu.VMEM((NP,), jnp.float32),
        pltpu.VMEM((_ZC,), jnp.float32),
        pltpu.VMEM_SHARED((NP * CW,), jnp.float32),
    ],
    compiler_params=_SC_PARAMS,
)
def _sc_coeff(src_hbm, tgt_hbm, et_hbm, invs_hbm, invt_hbm, cf_out, cb_out,
              gidx_v, nidx_v, eidx_v, fidx_v, vals_v, inv_v, stage_v, c_sh):
    s = lax.axis_index("s")
    c = lax.axis_index("c")

    def body(i, _):
        stage_v[pl.ds(i * 16, 16)] = jnp.zeros((16,), jnp.float32)
        return 0

    lax.fori_loop(0, _ZC // 16, body, 0)

    def run(val_hbm, val_idx_hbm, node_hbm, out_hbm, c_sh):
        pltpu.sync_copy(val_hbm, inv_v)
        cbase = s * _CPT
        for z in range(_CPT // _ZC):
            pltpu.sync_copy(stage_v, c_sh.at[pl.ds(cbase + z * _ZC, _ZC)])
        plsc.subcore_barrier()
        base = s * EPT

        def step(i, _):
            off = pl.multiple_of(base + i * K, K)
            pltpu.sync_copy(val_idx_hbm.at[pl.ds(off, K)], gidx_v)
            pltpu.sync_copy(node_hbm.at[pl.ds(off, K)], nidx_v)
            pltpu.sync_copy(et_hbm.at[pl.ds(off, K)], eidx_v)
            for j in range(K // 16):
                sl = pl.ds(j * 16, 16)
                vals_v[sl] = plsc.load_gather(inv_v, [gidx_v[sl]])
                fidx_v[sl] = nidx_v[sl] * CW + eidx_v[sl]
            pltpu.sync_copy(vals_v, c_sh.at[fidx_v], add=True)
            return 0

        lax.fori_loop(0, NCH, step, 0)
        plsc.subcore_barrier()
        for z in range(_CPT // _ZC):
            off = cbase + z * _ZC
            pltpu.sync_copy(c_sh.at[pl.ds(off, _ZC)], stage_v)
            pltpu.sync_copy(stage_v, out_hbm.at[pl.ds(off, _ZC)])

    @pl.when(c == 0)
    def _():
        run(invs_hbm, src_hbm, tgt_hbm, cf_out, c_sh)

    @pl.when(c == 1)
    def _():
        run(invt_hbm, tgt_hbm, src_hbm, cb_out, c_sh)


@functools.partial(
    pl.kernel,
    out_type=(_f32((NP, D)), _f32((NP, D))),
    mesh=_MESH,
    scratch_types=[
        pltpu.VMEM((K,), jnp.int32),
        pltpu.VMEM((K,), jnp.int32),
        pltpu.VMEM((K, D), jnp.float32),
        pltpu.VMEM_SHARED((NP, D), jnp.float32),
        pltpu.SemaphoreType.DMA,
    ],
    compiler_params=_SC_PARAMS,
)
def _sc_rows(xsf_hbm, xsb_hbm, src_hbm, tgt_hbm, zrows_hbm, accf_out, accb_out,
             gidx_v, sidx_v, rows_v, acc_sh, sem):
    s = lax.axis_index("s")
    c = lax.axis_index("c")

    def run(table_hbm, gather_hbm, scatter_hbm, out_hbm, acc_sh):
        pltpu.sync_copy(zrows_hbm, rows_v)
        for z in range(RPT // K):
            pltpu.sync_copy(rows_v, acc_sh.at[pl.ds(s * RPT + z * K, K)])
        plsc.subcore_barrier()
        base = s * EPT

        def step(i, _):
            off = pl.multiple_of(base + i * K, K)
            pltpu.sync_copy(gather_hbm.at[pl.ds(off, K)], gidx_v)
            pltpu.sync_copy(scatter_hbm.at[pl.ds(off, K)], sidx_v)
            pltpu.async_copy(table_hbm.at[gidx_v], rows_v, sem).wait()
            pltpu.sync_copy(rows_v, acc_sh.at[sidx_v], add=True)
            return 0

        lax.fori_loop(0, NCH, step, 0)
        plsc.subcore_barrier()
        for z in range(RPT // K):
            off = s * RPT + z * K
            pltpu.sync_copy(acc_sh.at[pl.ds(off, K)], rows_v)
            pltpu.sync_copy(rows_v, out_hbm.at[pl.ds(off, K)])

    @pl.when(c == 0)
    def _():
        run(xsf_hbm, src_hbm, tgt_hbm, accf_out, acc_sh)

    @pl.when(c == 1)
    def _():
        run(xsb_hbm, tgt_hbm, src_hbm, accb_out, acc_sh)


def _tc_prescale_body(x_ref, degs_ref, degt_ref, invs_ref, invt_ref,
                      xsf_ref, xsb_ref):
    invs = lax.rsqrt(jnp.maximum(degs_ref[...], 1.0))
    invt = lax.rsqrt(jnp.maximum(degt_ref[...], 1.0))
    invs_ref[...] = invs
    invt_ref[...] = invt
    x = x_ref[...]
    xsf_ref[...] = x * invs
    xsb_ref[...] = x * invt


def _tc_epilogue_body(x_ref, accf_ref, accb_ref, cf_ref, cb_ref, invs_ref,
                      invt_ref, xre_ref, xro_ref, wl_ref, wf_ref, wb_ref,
                      xr_ref, wr_ref, sl_ref, bias_ref, gamma_ref, beta_ref,
                      h_ref, xr_new_ref):
    f32 = jnp.float32
    relf = jnp.dot(cf_ref[...], xre_ref[...], preferred_element_type=f32)
    relb = jnp.dot(cb_ref[...], xro_ref[...], preferred_element_type=f32)
    fwd = jnp.dot(invt_ref[...] * (accf_ref[...] - relf), wf_ref[...],
                  preferred_element_type=f32)
    bwd = jnp.dot(invs_ref[...] * (accb_ref[...] - relb), wb_ref[...],
                  preferred_element_type=f32)
    loop = jnp.dot(x_ref[...] - sl_ref[...], wl_ref[...],
                   preferred_element_type=f32)
    h = (loop + fwd + bwd) * (1.0 / 3.0) + bias_ref[...]
    rows = lax.broadcasted_iota(jnp.int32, (NP, 1), 0)
    h = jnp.where(rows < N, h, 0.0)
    mean = jnp.sum(h, axis=0, keepdims=True) * (1.0 / N)
    var = jnp.sum(h * h, axis=0, keepdims=True) * (1.0 / N) - mean * mean
    inv_std = lax.rsqrt(var + 1e-5)
    h_ref[...] = (h - mean) * inv_std * gamma_ref[...] + beta_ref[...]
    xr_new_ref[...] = lax.dot_general(
        xr_ref[...], wr_ref[...], (((1,), (1,)), ((), ())),
        preferred_element_type=f32)


_tc_prescale = pl.pallas_call(
    _tc_prescale_body,
    out_shape=(_f32((NP, 1)), _f32((NP, 1)), _f32((NP, D)), _f32((NP, D))),
)

_tc_epilogue = pl.pallas_call(
    _tc_epilogue_body,
    out_shape=(_f32((NP, D)), _f32((200, D))),
)


def kernel(x_e, x_r, edge_index, edge_type, w_loop, w_fwd, w_bwd, w_rel,
           self_loop, bias, bn_gamma, bn_beta):
    pad = EPAD - E
    src = jnp.concatenate([edge_index[0], jnp.full((pad,), N, jnp.int32)])
    tgt = jnp.concatenate([edge_index[1], jnp.full((pad,), N, jnp.int32)])
    et = jnp.concatenate([edge_type, jnp.zeros((pad,), jnp.int32)])

    x_e_p = jnp.concatenate([x_e, jnp.zeros((NP - N, D), jnp.float32)])

    deg_s, deg_t = _sc_degrees(src, tgt)

    invs, invt, xs_f, xs_b = _tc_prescale(
        x_e_p, deg_s.reshape(NP, 1), deg_t.reshape(NP, 1))

    cf_flat, cb_flat = _sc_coeff(src, tgt, et, invs.reshape(NP),
                                 invt.reshape(NP))

    zrows = jnp.zeros((K, D), jnp.float32)
    acc_f, acc_b = _sc_rows(xs_f, xs_b, src, tgt, zrows)

    xr_even = jnp.concatenate([x_r[0::2], jnp.zeros((CW - NREL, D))])
    xr_odd = jnp.concatenate([x_r[1::2], jnp.zeros((CW - NREL, D))])

    h_full, x_r_new = _tc_epilogue(
        x_e_p, acc_f, acc_b, cf_flat.reshape(NP, CW), cb_flat.reshape(NP, CW),
        invs, invt, xr_even, xr_odd, w_loop, w_fwd, w_bwd, x_r, w_rel,
        self_loop, bias.reshape(1, D), bn_gamma.reshape(1, D),
        bn_beta.reshape(1, D))

    return h_full[:N], x_r_new


# speedup vs baseline: 7.0173x; 1.2304x over previous
import functools

import jax
import jax.numpy as jnp
from jax import lax
from jax.experimental import pallas as pl
from jax.experimental.pallas import tpu as pltpu
from jax.experimental.pallas import tpu_sc as plsc

N = 10000
E = 320000
D = 128
NREL = 100
CW = 128

NP = 10240
NSUB = 16
K = 128
EPT = 20480
EPAD = NSUB * EPT
NCH = EPT // K
RPT = NP // NSUB

_MESH = plsc.VectorSubcoreMesh(core_axis_name="c", subcore_axis_name="s")
_SC_PARAMS = pltpu.CompilerParams(needs_layout_passes=False)


def _f32(shape):
    return jax.ShapeDtypeStruct(shape, jnp.float32)


@functools.partial(
    pl.kernel,
    out_type=(_f32((NP,)), _f32((NP,))),
    mesh=_MESH,
    scratch_types=[
        pltpu.VMEM((K,), jnp.int32),
        pltpu.VMEM((K,), jnp.float32),
        pltpu.VMEM((RPT,), jnp.float32),
        pltpu.VMEM_SHARED((NP,), jnp.float32),
    ],
    compiler_params=_SC_PARAMS,
)
def _sc_degrees(src_hbm, tgt_hbm, degs_out, degt_out, idx_v, ones_v, row_v,
                acc_sh):
    s = lax.axis_index("s")
    c = lax.axis_index("c")

    def fill(ref, n, value):
        def body(i, _):
            ref[pl.ds(i * 16, 16)] = jnp.full((16,), value, jnp.float32)
            return 0
        lax.fori_loop(0, n // 16, body, 0)

    def run(eidx_hbm, out_hbm, acc_sh):
        fill(ones_v, K, 1.0)
        fill(row_v, RPT, 0.0)
        pltpu.sync_copy(row_v, acc_sh.at[pl.ds(s * RPT, RPT)])
        plsc.subcore_barrier()
        base = s * EPT

        def step(i, _):
            off = pl.multiple_of(base + i * K, K)
            pltpu.sync_copy(eidx_hbm.at[pl.ds(off, K)], idx_v)
            pltpu.sync_copy(ones_v, acc_sh.at[idx_v], add=True)
            return 0

        lax.fori_loop(0, NCH, step, 0)
        plsc.subcore_barrier()
        pltpu.sync_copy(acc_sh.at[pl.ds(s * RPT, RPT)], row_v)
        pltpu.sync_copy(row_v, out_hbm.at[pl.ds(s * RPT, RPT)])

    @pl.when(c == 0)
    def _():
        run(src_hbm, degs_out, acc_sh)

    @pl.when(c == 1)
    def _():
        run(tgt_hbm, degt_out, acc_sh)


_CPT = NP * CW // NSUB
_ZC = 8192

@functools.partial(
    pl.kernel,
    out_type=(_f32((NP * CW,)), _f32((NP * CW,))),
    mesh=_MESH,
    scratch_types=[
        pltpu.VMEM((K,), jnp.int32),
        pltpu.VMEM((K,), jnp.int32),
        pltpu.VMEM((K,), jnp.int32),
        pltpu.VMEM((K,), jnp.int32),
        pltpu.VMEM((K,), jnp.float32),
        pltpu.VMEM((NP,), jnp.float32),
        pltpu.VMEM((_ZC,), jnp.float32),
        pltpu.VMEM_SHARED((NP * CW,), jnp.float32),
    ],
    compiler_params=_SC_PARAMS,
)
def _sc_coeff(src_hbm, tgt_hbm, et_hbm, invs_hbm, invt_hbm, cf_out, cb_out,
              gidx_v, nidx_v, eidx_v, fidx_v, vals_v, inv_v, stage_v, c_sh):
    s = lax.axis_index("s")
    c = lax.axis_index("c")

    def body(i, _):
        stage_v[pl.ds(i * 16, 16)] = jnp.zeros((16,), jnp.float32)
        return 0

    lax.fori_loop(0, _ZC // 16, body, 0)

    def run(val_hbm, val_idx_hbm, node_hbm, out_hbm, c_sh):
        pltpu.sync_copy(val_hbm, inv_v)
        cbase = s * _CPT
        for z in range(_CPT // _ZC):
            pltpu.sync_copy(stage_v, c_sh.at[pl.ds(cbase + z * _ZC, _ZC)])
        plsc.subcore_barrier()
        base = s * EPT

        def step(i, _):
            off = pl.multiple_of(base + i * K, K)
            pltpu.sync_copy(val_idx_hbm.at[pl.ds(off, K)], gidx_v)
            pltpu.sync_copy(node_hbm.at[pl.ds(off, K)], nidx_v)
            pltpu.sync_copy(et_hbm.at[pl.ds(off, K)], eidx_v)
            for j in range(K // 16):
                sl = pl.ds(j * 16, 16)
                vals_v[sl] = plsc.load_gather(inv_v, [gidx_v[sl]])
                fidx_v[sl] = nidx_v[sl] * CW + eidx_v[sl]
            pltpu.sync_copy(vals_v, c_sh.at[fidx_v], add=True)
            return 0

        lax.fori_loop(0, NCH, step, 0)
        plsc.subcore_barrier()
        for z in range(_CPT // _ZC):
            off = cbase + z * _ZC
            pltpu.sync_copy(c_sh.at[pl.ds(off, _ZC)], stage_v)
            pltpu.sync_copy(stage_v, out_hbm.at[pl.ds(off, _ZC)])

    @pl.when(c == 0)
    def _():
        run(invs_hbm, src_hbm, tgt_hbm, cf_out, c_sh)

    @pl.when(c == 1)
    def _():
        run(invt_hbm, tgt_hbm, src_hbm, cb_out, c_sh)


KB = 64
NCHB = EPT // KB
NROW = 4
NIDX = 8
RCH = RPT // KB

@functools.partial(
    pl.kernel,
    out_type=(_f32((NP, D)), _f32((NP, D))),
    mesh=_MESH,
    scratch_types=[
        *([pltpu.VMEM((2, KB), jnp.int32)] * NIDX),
        *([pltpu.VMEM((KB, D), jnp.float32)] * NROW),
        pltpu.VMEM_SHARED((NP, D), jnp.float32),
        *([pltpu.SemaphoreType.DMA] * (NROW + NROW + NIDX)),
    ],
    compiler_params=_SC_PARAMS,
)
def _sc_rows(xsf_hbm, xsb_hbm, pairs_hbm, zrows_hbm, accf_out, accb_out,
             *rest):
    ibuf = rest[:NIDX]
    rows = rest[NIDX:NIDX + NROW]
    acc_sh = rest[NIDX + NROW]
    gsem = rest[NIDX + NROW + 1:NIDX + NROW + 1 + NROW]
    ssem = rest[NIDX + NROW + 1 + NROW:NIDX + NROW + 1 + 2 * NROW]
    isem = rest[NIDX + NROW + 1 + 2 * NROW:]
    s = lax.axis_index("s")
    c = lax.axis_index("c")

    def run(table_hbm, gi, si, out_hbm):
        pltpu.sync_copy(zrows_hbm, rows[0])
        for z in range(RCH):
            pltpu.sync_copy(rows[0], acc_sh.at[pl.ds(s * RPT + z * KB, KB)])
        plsc.subcore_barrier()

        def idxf(chunk, j):
            return pltpu.async_copy(pairs_hbm.at[s].at[chunk], ibuf[j],
                                    isem[j])

        def gather(j, b):
            return pltpu.async_copy(table_hbm.at[ibuf[j].at[gi]], rows[b],
                                    gsem[b])

        def scatter(j, b):
            return pltpu.async_copy(rows[b], acc_sh.at[ibuf[j].at[si]],
                                    ssem[b], add=True)

        cps = [idxf(cc, cc) for cc in range(4)]
        for cc in range(2):
            cps[cc].wait()
            gather(cc, cc)

        def step(k, _):
            for j in range(NIDX):
                chunk = k * NIDX + j
                b = j % NROW
                pltpu.make_async_copy(table_hbm.at[ibuf[j].at[gi]], rows[b],
                                      gsem[b]).wait()
                scatter(j, b)
                j2 = (j + 2) % NIDX
                b2 = (j + 2) % NROW

                @pl.when(chunk + 2 < NCHB)
                def _():
                    @pl.when(chunk >= 2)
                    def _():
                        pltpu.make_async_copy(
                            rows[b2], acc_sh.at[ibuf[j2].at[si]],
                            ssem[b2]).wait()
                    pltpu.make_async_copy(pairs_hbm.at[s].at[chunk],
                                          ibuf[j2], isem[j2]).wait()
                    gather(j2, b2)

                j4 = (j + 4) % NIDX

                @pl.when(chunk + 4 < NCHB)
                def _():
                    idxf(chunk + 4, j4)
            return 0

        lax.fori_loop(0, NCHB // NIDX, step, 0)
        for chunk in range(NCHB - NROW, NCHB):
            j, b = chunk % NIDX, chunk % NROW
            pltpu.make_async_copy(rows[b], acc_sh.at[ibuf[j].at[si]],
                                  ssem[b]).wait()
        plsc.subcore_barrier()
        for z in range(RCH):
            off = s * RPT + z * KB
            pltpu.sync_copy(acc_sh.at[pl.ds(off, KB)], rows[0])
            pltpu.sync_copy(rows[0], out_hbm.at[pl.ds(off, KB)])

    @pl.when(c == 0)
    def _():
        run(xsf_hbm, 0, 1, accf_out)

    @pl.when(c == 1)
    def _():
        run(xsb_hbm, 1, 0, accb_out)


def _tc_prescale_body(x_ref, degs_ref, degt_ref, invs_ref, invt_ref,
                      xsf_ref, xsb_ref):
    invs = lax.rsqrt(jnp.maximum(degs_ref[...], 1.0))
    invt = lax.rsqrt(jnp.maximum(degt_ref[...], 1.0))
    invs_ref[...] = invs
    invt_ref[...] = invt
    x = x_ref[...]
    xsf_ref[...] = x * invs
    xsb_ref[...] = x * invt


def _tc_epilogue_body(x_ref, accf_ref, accb_ref, cf_ref, cb_ref, invs_ref,
                      invt_ref, xre_ref, xro_ref, wl_ref, wf_ref, wb_ref,
                      xr_ref, wr_ref, sl_ref, bias_ref, gamma_ref, beta_ref,
                      h_ref, xr_new_ref):
    f32 = jnp.float32
    relf = jnp.dot(cf_ref[...], xre_ref[...], preferred_element_type=f32)
    relb = jnp.dot(cb_ref[...], xro_ref[...], preferred_element_type=f32)
    fwd = jnp.dot(invt_ref[...] * (accf_ref[...] - relf), wf_ref[...],
                  preferred_element_type=f32)
    bwd = jnp.dot(invs_ref[...] * (accb_ref[...] - relb), wb_ref[...],
                  preferred_element_type=f32)
    loop = jnp.dot(x_ref[...] - sl_ref[...], wl_ref[...],
                   preferred_element_type=f32)
    h = (loop + fwd + bwd) * (1.0 / 3.0) + bias_ref[...]
    rows = lax.broadcasted_iota(jnp.int32, (NP, 1), 0)
    h = jnp.where(rows < N, h, 0.0)
    mean = jnp.sum(h, axis=0, keepdims=True) * (1.0 / N)
    var = jnp.sum(h * h, axis=0, keepdims=True) * (1.0 / N) - mean * mean
    inv_std = lax.rsqrt(var + 1e-5)
    h_ref[...] = (h - mean) * inv_std * gamma_ref[...] + beta_ref[...]
    xr_new_ref[...] = lax.dot_general(
        xr_ref[...], wr_ref[...], (((1,), (1,)), ((), ())),
        preferred_element_type=f32)


_tc_prescale = pl.pallas_call(
    _tc_prescale_body,
    out_shape=(_f32((NP, 1)), _f32((NP, 1)), _f32((NP, D)), _f32((NP, D))),
)

_tc_epilogue = pl.pallas_call(
    _tc_epilogue_body,
    out_shape=(_f32((NP, D)), _f32((200, D))),
)


def kernel(x_e, x_r, edge_index, edge_type, w_loop, w_fwd, w_bwd, w_rel,
           self_loop, bias, bn_gamma, bn_beta):
    pad = EPAD - E
    src = jnp.concatenate([edge_index[0], jnp.full((pad,), N, jnp.int32)])
    tgt = jnp.concatenate([edge_index[1], jnp.full((pad,), N, jnp.int32)])
    et = jnp.concatenate([edge_type, jnp.zeros((pad,), jnp.int32)])

    x_e_p = jnp.concatenate([x_e, jnp.zeros((NP - N, D), jnp.float32)])

    deg_s, deg_t = _sc_degrees(src, tgt)

    invs, invt, xs_f, xs_b = _tc_prescale(
        x_e_p, deg_s.reshape(NP, 1), deg_t.reshape(NP, 1))

    cf_flat, cb_flat = _sc_coeff(src, tgt, et, invs.reshape(NP),
                                 invt.reshape(NP))

    zrows = jnp.zeros((KB, D), jnp.float32)
    pairs = jnp.stack([src.reshape(NSUB, NCHB, KB),
                       tgt.reshape(NSUB, NCHB, KB)], axis=2)
    acc_f, acc_b = _sc_rows(xs_f, xs_b, pairs, zrows)

    xr_even = jnp.concatenate([x_r[0::2], jnp.zeros((CW - NREL, D))])
    xr_odd = jnp.concatenate([x_r[1::2], jnp.zeros((CW - NREL, D))])

    h_full, x_r_new = _tc_epilogue(
        x_e_p, acc_f, acc_b, cf_flat.reshape(NP, CW), cb_flat.reshape(NP, CW),
        invs, invt, xr_even, xr_odd, w_loop, w_fwd, w_bwd, x_r, w_rel,
        self_loop, bias.reshape(1, D), bn_gamma.reshape(1, D),
        bn_beta.reshape(1, D))

    return h_full[:N], x_r_new


# speedup vs baseline: 9.3220x; 1.3284x over previous
import functools

import jax
import jax.numpy as jnp
from jax import lax
from jax.experimental import pallas as pl
from jax.experimental.pallas import tpu as pltpu
from jax.experimental.pallas import tpu_sc as plsc

N = 10000
E = 320000
D = 128
NREL = 100
CW = 128

NP = 10240
NSUB = 16
K = 128
EPT = 20480
EPAD = NSUB * EPT
NCH = EPT // K
RPT = NP // NSUB

_MESH = plsc.VectorSubcoreMesh(core_axis_name="c", subcore_axis_name="s")
_SC_PARAMS = pltpu.CompilerParams(needs_layout_passes=False)


def _f32(shape):
    return jax.ShapeDtypeStruct(shape, jnp.float32)


KB = 64
NCHB = EPT // KB
NIDXA = 8

@functools.partial(
    pl.kernel,
    out_type=(_f32((NP,)), _f32((NP,))),
    mesh=_MESH,
    scratch_types=[
        *([pltpu.VMEM((2, KB), jnp.int32)] * NIDXA),
        pltpu.VMEM((KB,), jnp.float32),
        pltpu.VMEM((RPT,), jnp.float32),
        pltpu.VMEM_SHARED((NP,), jnp.float32),
        *([pltpu.SemaphoreType.DMA] * (2 * NIDXA)),
    ],
    compiler_params=_SC_PARAMS,
)
def _sc_degrees(pairs_hbm, degs_out, degt_out, *rest):
    ibuf = rest[:NIDXA]
    ones_v = rest[NIDXA]
    row_v = rest[NIDXA + 1]
    acc_sh = rest[NIDXA + 2]
    isem = rest[NIDXA + 3:NIDXA + 3 + NIDXA]
    ssem = rest[NIDXA + 3 + NIDXA:]
    s = lax.axis_index("s")
    c = lax.axis_index("c")

    def fill(ref, n, value):
        def body(i, _):
            ref[pl.ds(i * 16, 16)] = jnp.full((16,), value, jnp.float32)
            return 0
        lax.fori_loop(0, n // 16, body, 0)

    def run(gi, out_hbm):
        fill(ones_v, KB, 1.0)
        fill(row_v, RPT, 0.0)
        pltpu.sync_copy(row_v, acc_sh.at[pl.ds(s * RPT, RPT)])
        plsc.subcore_barrier()

        def idxf(chunk, j):
            return pltpu.async_copy(pairs_hbm.at[s].at[chunk], ibuf[j],
                                    isem[j])

        def scatter(j):
            return pltpu.async_copy(ones_v, acc_sh.at[ibuf[j].at[gi]],
                                    ssem[j], add=True)

        for cc in range(4):
            idxf(cc, cc)

        def step(k, _):
            for j in range(NIDXA):
                chunk = k * NIDXA + j
                pltpu.make_async_copy(pairs_hbm.at[s].at[chunk], ibuf[j],
                                      isem[j]).wait()
                scatter(j)
                j4 = (j + 4) % NIDXA

                @pl.when(chunk + 4 < NCHB)
                def _():
                    @pl.when(chunk >= 4)
                    def _():
                        pltpu.make_async_copy(
                            ones_v, acc_sh.at[ibuf[j4].at[gi]],
                            ssem[j4]).wait()
                    idxf(chunk + 4, j4)
            return 0

        lax.fori_loop(0, NCHB // NIDXA, step, 0)
        for chunk in range(NCHB - NIDXA, NCHB):
            j = chunk % NIDXA
            pltpu.make_async_copy(ones_v, acc_sh.at[ibuf[j].at[gi]],
                                  ssem[j]).wait()
        plsc.subcore_barrier()
        pltpu.sync_copy(acc_sh.at[pl.ds(s * RPT, RPT)], row_v)
        pltpu.sync_copy(row_v, out_hbm.at[pl.ds(s * RPT, RPT)])

    @pl.when(c == 0)
    def _():
        run(0, degs_out)

    @pl.when(c == 1)
    def _():
        run(1, degt_out)


_CPT = NP * CW // NSUB
_ZC = 8192
NIDXC = 8
NRC = 4

@functools.partial(
    pl.kernel,
    out_type=(_f32((NP * CW,)), _f32((NP * CW,))),
    mesh=_MESH,
    scratch_types=[
        *([pltpu.VMEM((3, K), jnp.int32)] * NIDXC),
        *([pltpu.VMEM((K,), jnp.int32)] * NRC),
        *([pltpu.VMEM((K,), jnp.float32)] * NRC),
        pltpu.VMEM((NP,), jnp.float32),
        pltpu.VMEM((_ZC,), jnp.float32),
        pltpu.VMEM_SHARED((NP * CW,), jnp.float32),
        *([pltpu.SemaphoreType.DMA] * (NIDXC + NRC)),
    ],
    compiler_params=_SC_PARAMS,
)
def _sc_coeff(triple_hbm, invs_hbm, invt_hbm, cf_out, cb_out, *rest):
    ibuf = rest[:NIDXC]
    fidx = rest[NIDXC:NIDXC + NRC]
    vals = rest[NIDXC + NRC:NIDXC + 2 * NRC]
    inv_v = rest[NIDXC + 2 * NRC]
    stage_v = rest[NIDXC + 2 * NRC + 1]
    c_sh = rest[NIDXC + 2 * NRC + 2]
    isem = rest[NIDXC + 2 * NRC + 3:NIDXC + 2 * NRC + 3 + NIDXC]
    ssem = rest[NIDXC + 2 * NRC + 3 + NIDXC:]
    s = lax.axis_index("s")
    c = lax.axis_index("c")

    def zero(i, _):
        stage_v[pl.ds(i * 16, 16)] = jnp.zeros((16,), jnp.float32)
        return 0

    lax.fori_loop(0, _ZC // 16, zero, 0)

    def run(val_hbm, gi, si, out_hbm):
        pltpu.sync_copy(val_hbm, inv_v)
        cbase = s * _CPT
        for z in range(_CPT // _ZC):
            pltpu.sync_copy(stage_v, c_sh.at[pl.ds(cbase + z * _ZC, _ZC)])
        plsc.subcore_barrier()

        def idxf(chunk, j):
            return pltpu.async_copy(triple_hbm.at[s].at[chunk], ibuf[j],
                                    isem[j])

        for cc in range(4):
            idxf(cc, cc)

        def step(k, _):
            for j in range(NIDXC):
                chunk = k * NIDXC + j
                b = j % NRC
                pltpu.make_async_copy(triple_hbm.at[s].at[chunk], ibuf[j],
                                      isem[j]).wait()

                @pl.when(chunk >= NRC)
                def _():
                    pltpu.make_async_copy(vals[b], c_sh.at[fidx[b]],
                                          ssem[b]).wait()

                for g in range(K // 16):
                    sl = pl.ds(g * 16, 16)
                    vals[b][sl] = plsc.load_gather(inv_v, [ibuf[j][gi, sl]])
                    fidx[b][sl] = ibuf[j][si, sl] * CW + ibuf[j][2, sl]
                pltpu.async_copy(vals[b], c_sh.at[fidx[b]], ssem[b],
                                 add=True)
                j4 = (j + 4) % NIDXC

                @pl.when(chunk + 4 < NCH)
                def _():
                    idxf(chunk + 4, j4)
            return 0

        lax.fori_loop(0, NCH // NIDXC, step, 0)
        for chunk in range(NCH - NRC, NCH):
            b = chunk % NRC
            pltpu.make_async_copy(vals[b], c_sh.at[fidx[b]], ssem[b]).wait()
        plsc.subcore_barrier()
        for z in range(_CPT // _ZC):
            off = cbase + z * _ZC
            pltpu.sync_copy(c_sh.at[pl.ds(off, _ZC)], stage_v)
            pltpu.sync_copy(stage_v, out_hbm.at[pl.ds(off, _ZC)])

    @pl.when(c == 0)
    def _():
        run(invs_hbm, 0, 1, cf_out)

    @pl.when(c == 1)
    def _():
        run(invt_hbm, 1, 0, cb_out)


NROW = 5
NIDX = 10
LOOK = 3
RCH = RPT // KB

@functools.partial(
    pl.kernel,
    out_type=(_f32((NP, D)), _f32((NP, D))),
    mesh=_MESH,
    scratch_types=[
        *([pltpu.VMEM((2, KB), jnp.int32)] * NIDX),
        *([pltpu.VMEM((KB, D), jnp.float32)] * NROW),
        pltpu.VMEM_SHARED((NP, D), jnp.float32),
        *([pltpu.SemaphoreType.DMA] * (NROW + NROW + NIDX)),
    ],
    compiler_params=_SC_PARAMS,
)
def _sc_rows(xsf_hbm, xsb_hbm, pairs_hbm, zrows_hbm, accf_out, accb_out,
             *rest):
    ibuf = rest[:NIDX]
    rows = rest[NIDX:NIDX + NROW]
    acc_sh = rest[NIDX + NROW]
    gsem = rest[NIDX + NROW + 1:NIDX + NROW + 1 + NROW]
    ssem = rest[NIDX + NROW + 1 + NROW:NIDX + NROW + 1 + 2 * NROW]
    isem = rest[NIDX + NROW + 1 + 2 * NROW:]
    s = lax.axis_index("s")
    c = lax.axis_index("c")

    def run(table_hbm, gi, si, out_hbm):
        pltpu.sync_copy(zrows_hbm, rows[0])
        for z in range(RCH):
            pltpu.sync_copy(rows[0], acc_sh.at[pl.ds(s * RPT + z * KB, KB)])
        plsc.subcore_barrier()

        def idxf(chunk, j):
            return pltpu.async_copy(pairs_hbm.at[s].at[chunk], ibuf[j],
                                    isem[j])

        def gather(j, b):
            return pltpu.async_copy(table_hbm.at[ibuf[j].at[gi]], rows[b],
                                    gsem[b])

        def scatter(j, b):
            return pltpu.async_copy(rows[b], acc_sh.at[ibuf[j].at[si]],
                                    ssem[b], add=True)

        cps = [idxf(cc, cc) for cc in range(NROW)]
        for cc in range(LOOK):
            cps[cc].wait()
            gather(cc, cc)

        def step(k, _):
            for j in range(NIDX):
                chunk = k * NIDX + j
                b = j % NROW
                pltpu.make_async_copy(table_hbm.at[ibuf[j].at[gi]], rows[b],
                                      gsem[b]).wait()
                scatter(j, b)
                j2 = (j + LOOK) % NIDX
                b2 = (j + LOOK) % NROW

                @pl.when(chunk + LOOK < NCHB)
                def _():
                    @pl.when(chunk >= NROW - LOOK)
                    def _():
                        pltpu.make_async_copy(
                            rows[b2], acc_sh.at[ibuf[j2].at[si]],
                            ssem[b2]).wait()
                    pltpu.make_async_copy(pairs_hbm.at[s].at[chunk],
                                          ibuf[j2], isem[j2]).wait()
                    gather(j2, b2)

                j4 = (j + NROW) % NIDX

                @pl.when(chunk + NROW < NCHB)
                def _():
                    idxf(chunk + NROW, j4)
            return 0

        lax.fori_loop(0, NCHB // NIDX, step, 0)
        for chunk in range(NCHB - NROW, NCHB):
            j, b = chunk % NIDX, chunk % NROW
            pltpu.make_async_copy(rows[b], acc_sh.at[ibuf[j].at[si]],
                                  ssem[b]).wait()
        plsc.subcore_barrier()
        for z in range(RCH):
            off = s * RPT + z * KB
            pltpu.sync_copy(acc_sh.at[pl.ds(off, KB)], rows[0])
            pltpu.sync_copy(rows[0], out_hbm.at[pl.ds(off, KB)])

    @pl.when(c == 0)
    def _():
        run(xsf_hbm, 0, 1, accf_out)

    @pl.when(c == 1)
    def _():
        run(xsb_hbm, 1, 0, accb_out)


def _tc_prescale_body(x_ref, degs_ref, degt_ref, invs_ref, invt_ref,
                      xsf_ref, xsb_ref):
    invs = lax.rsqrt(jnp.maximum(degs_ref[...], 1.0))
    invt = lax.rsqrt(jnp.maximum(degt_ref[...], 1.0))
    invs_ref[...] = invs
    invt_ref[...] = invt
    x = x_ref[...]
    xsf_ref[...] = x * invs
    xsb_ref[...] = x * invt


def _tc_epilogue_body(x_ref, accf_ref, accb_ref, cf_ref, cb_ref, invs_ref,
                      invt_ref, xre_ref, xro_ref, wl_ref, wf_ref, wb_ref,
                      xr_ref, wr_ref, sl_ref, bias_ref, gamma_ref, beta_ref,
                      h_ref, xr_new_ref):
    f32 = jnp.float32
    relf = jnp.dot(cf_ref[...], xre_ref[...], preferred_element_type=f32)
    relb = jnp.dot(cb_ref[...], xro_ref[...], preferred_element_type=f32)
    fwd = jnp.dot(invt_ref[...] * (accf_ref[...] - relf), wf_ref[...],
                  preferred_element_type=f32)
    bwd = jnp.dot(invs_ref[...] * (accb_ref[...] - relb), wb_ref[...],
                  preferred_element_type=f32)
    loop = jnp.dot(x_ref[...] - sl_ref[...], wl_ref[...],
                   preferred_element_type=f32)
    h = (loop + fwd + bwd) * (1.0 / 3.0) + bias_ref[...]
    rows = lax.broadcasted_iota(jnp.int32, (NP, 1), 0)
    h = jnp.where(rows < N, h, 0.0)
    mean = jnp.sum(h, axis=0, keepdims=True) * (1.0 / N)
    var = jnp.sum(h * h, axis=0, keepdims=True) * (1.0 / N) - mean * mean
    inv_std = lax.rsqrt(var + 1e-5)
    h_ref[...] = (h - mean) * inv_std * gamma_ref[...] + beta_ref[...]
    xr_new_ref[...] = lax.dot_general(
        xr_ref[...], wr_ref[...], (((1,), (1,)), ((), ())),
        preferred_element_type=f32)


_tc_prescale = pl.pallas_call(
    _tc_prescale_body,
    out_shape=(_f32((NP, 1)), _f32((NP, 1)), _f32((NP, D)), _f32((NP, D))),
)

_tc_epilogue = pl.pallas_call(
    _tc_epilogue_body,
    out_shape=(_f32((NP, D)), _f32((200, D))),
)


def kernel(x_e, x_r, edge_index, edge_type, w_loop, w_fwd, w_bwd, w_rel,
           self_loop, bias, bn_gamma, bn_beta):
    pad = EPAD - E
    src = jnp.concatenate([edge_index[0], jnp.full((pad,), N, jnp.int32)])
    tgt = jnp.concatenate([edge_index[1], jnp.full((pad,), N, jnp.int32)])
    et = jnp.concatenate([edge_type, jnp.zeros((pad,), jnp.int32)])

    x_e_p = jnp.concatenate([x_e, jnp.zeros((NP - N, D), jnp.float32)])

    pairs = jnp.stack([src.reshape(NSUB, NCHB, KB),
                       tgt.reshape(NSUB, NCHB, KB)], axis=2)
    triple = jnp.stack([src.reshape(NSUB, NCH, K),
                        tgt.reshape(NSUB, NCH, K),
                        et.reshape(NSUB, NCH, K)], axis=2)

    deg_s, deg_t = _sc_degrees(pairs)

    invs, invt, xs_f, xs_b = _tc_prescale(
        x_e_p, deg_s.reshape(NP, 1), deg_t.reshape(NP, 1))

    cf_flat, cb_flat = _sc_coeff(triple, invs.reshape(NP), invt.reshape(NP))

    zrows = jnp.zeros((KB, D), jnp.float32)
    acc_f, acc_b = _sc_rows(xs_f, xs_b, pairs, zrows)

    xr_even = jnp.concatenate([x_r[0::2], jnp.zeros((CW - NREL, D))])
    xr_odd = jnp.concatenate([x_r[1::2], jnp.zeros((CW - NREL, D))])

    h_full, x_r_new = _tc_epilogue(
        x_e_p, acc_f, acc_b, cf_flat.reshape(NP, CW), cb_flat.reshape(NP, CW),
        invs, invt, xr_even, xr_odd, w_loop, w_fwd, w_bwd, x_r, w_rel,
        self_loop, bias.reshape(1, D), bn_gamma.reshape(1, D),
        bn_beta.reshape(1, D))

    return h_full[:N], x_r_new


# speedup vs baseline: 20.7924x; 2.2305x over previous
import functools

import jax
import jax.numpy as jnp
from jax import lax
from jax.experimental import pallas as pl
from jax.experimental.pallas import tpu as pltpu
from jax.experimental.pallas import tpu_sc as plsc

N = 10000
E = 320000
D = 128
NREL = 100
CW = 128

NP = 10240
NSUB = 16
K = 128
EPT = 20480
EPAD = NSUB * EPT
NCH = EPT // K
RPT = NP // NSUB

_MESH = plsc.VectorSubcoreMesh(core_axis_name="c", subcore_axis_name="s")
_SC_PARAMS = pltpu.CompilerParams(needs_layout_passes=False)


def _f32(shape):
    return jax.ShapeDtypeStruct(shape, jnp.float32)


KB = 64
NCHB = EPT // KB
NIDXA = 8

@functools.partial(
    pl.kernel,
    out_type=(_f32((NP,)), _f32((NP,))),
    mesh=_MESH,
    scratch_types=[
        *([pltpu.VMEM((2, KB), jnp.int32)] * NIDXA),
        pltpu.VMEM((KB,), jnp.float32),
        pltpu.VMEM((RPT,), jnp.float32),
        pltpu.VMEM_SHARED((NP,), jnp.float32),
        *([pltpu.SemaphoreType.DMA] * (2 * NIDXA)),
    ],
    compiler_params=_SC_PARAMS,
)
def _sc_degrees(pairs_hbm, degs_out, degt_out, *rest):
    ibuf = rest[:NIDXA]
    ones_v = rest[NIDXA]
    row_v = rest[NIDXA + 1]
    acc_sh = rest[NIDXA + 2]
    isem = rest[NIDXA + 3:NIDXA + 3 + NIDXA]
    ssem = rest[NIDXA + 3 + NIDXA:]
    s = lax.axis_index("s")
    c = lax.axis_index("c")

    def fill(ref, n, value):
        def body(i, _):
            ref[pl.ds(i * 16, 16)] = jnp.full((16,), value, jnp.float32)
            return 0
        lax.fori_loop(0, n // 16, body, 0)

    def run(gi, out_hbm):
        fill(ones_v, KB, 1.0)
        fill(row_v, RPT, 0.0)
        pltpu.sync_copy(row_v, acc_sh.at[pl.ds(s * RPT, RPT)])
        plsc.subcore_barrier()

        def idxf(chunk, j):
            return pltpu.async_copy(pairs_hbm.at[s].at[chunk], ibuf[j],
                                    isem[j])

        def scatter(j):
            return pltpu.async_copy(ones_v, acc_sh.at[ibuf[j].at[gi]],
                                    ssem[j], add=True)

        for cc in range(4):
            idxf(cc, cc)

        def step(k, _):
            for j in range(NIDXA):
                chunk = k * NIDXA + j
                pltpu.make_async_copy(pairs_hbm.at[s].at[chunk], ibuf[j],
                                      isem[j]).wait()
                scatter(j)
                j4 = (j + 4) % NIDXA

                @pl.when(chunk + 4 < NCHB)
                def _():
                    @pl.when(chunk >= 4)
                    def _():
                        pltpu.make_async_copy(
                            ones_v, acc_sh.at[ibuf[j4].at[gi]],
                            ssem[j4]).wait()
                    idxf(chunk + 4, j4)
            return 0

        lax.fori_loop(0, NCHB // NIDXA, step, 0)
        for chunk in range(NCHB - NIDXA, NCHB):
            j = chunk % NIDXA
            pltpu.make_async_copy(ones_v, acc_sh.at[ibuf[j].at[gi]],
                                  ssem[j]).wait()
        plsc.subcore_barrier()
        pltpu.sync_copy(acc_sh.at[pl.ds(s * RPT, RPT)], row_v)
        pltpu.sync_copy(row_v, out_hbm.at[pl.ds(s * RPT, RPT)])

    @pl.when(c == 0)
    def _():
        run(0, degs_out)

    @pl.when(c == 1)
    def _():
        run(1, degt_out)


_CPT = NP * CW // NSUB
_ZC = 8192
NIDXC = 8
NRC = 4

@functools.partial(
    pl.kernel,
    out_type=(_f32((NP * CW,)), _f32((NP * CW,))),
    mesh=_MESH,
    scratch_types=[
        *([pltpu.VMEM((3, K), jnp.int32)] * NIDXC),
        *([pltpu.VMEM((K,), jnp.int32)] * NRC),
        *([pltpu.VMEM((K,), jnp.float32)] * NRC),
        pltpu.VMEM((NP,), jnp.float32),
        pltpu.VMEM((_ZC,), jnp.float32),
        pltpu.VMEM_SHARED((NP * CW,), jnp.float32),
        *([pltpu.SemaphoreType.DMA] * (NIDXC + NRC)),
    ],
    compiler_params=_SC_PARAMS,
)
def _sc_coeff(triple_hbm, invs_hbm, invt_hbm, cf_out, cb_out, *rest):
    ibuf = rest[:NIDXC]
    fidx = rest[NIDXC:NIDXC + NRC]
    vals = rest[NIDXC + NRC:NIDXC + 2 * NRC]
    inv_v = rest[NIDXC + 2 * NRC]
    stage_v = rest[NIDXC + 2 * NRC + 1]
    c_sh = rest[NIDXC + 2 * NRC + 2]
    isem = rest[NIDXC + 2 * NRC + 3:NIDXC + 2 * NRC + 3 + NIDXC]
    ssem = rest[NIDXC + 2 * NRC + 3 + NIDXC:]
    s = lax.axis_index("s")
    c = lax.axis_index("c")

    def zero(i, _):
        stage_v[pl.ds(i * 16, 16)] = jnp.zeros((16,), jnp.float32)
        return 0

    lax.fori_loop(0, _ZC // 16, zero, 0)

    def run(val_hbm, gi, si, out_hbm):
        pltpu.sync_copy(val_hbm, inv_v)
        cbase = s * _CPT
        for z in range(_CPT // _ZC):
            pltpu.sync_copy(stage_v, c_sh.at[pl.ds(cbase + z * _ZC, _ZC)])
        plsc.subcore_barrier()

        def idxf(chunk, j):
            return pltpu.async_copy(triple_hbm.at[s].at[chunk], ibuf[j],
                                    isem[j])

        for cc in range(4):
            idxf(cc, cc)

        def step(k, _):
            for j in range(NIDXC):
                chunk = k * NIDXC + j
                b = j % NRC
                pltpu.make_async_copy(triple_hbm.at[s].at[chunk], ibuf[j],
                                      isem[j]).wait()

                @pl.when(chunk >= NRC)
                def _():
                    pltpu.make_async_copy(vals[b], c_sh.at[fidx[b]],
                                          ssem[b]).wait()

                for g in range(K // 16):
                    sl = pl.ds(g * 16, 16)
                    vals[b][sl] = plsc.load_gather(inv_v, [ibuf[j][gi, sl]])
                    fidx[b][sl] = ibuf[j][si, sl] * CW + ibuf[j][2, sl]
                pltpu.async_copy(vals[b], c_sh.at[fidx[b]], ssem[b],
                                 add=True)
                j4 = (j + 4) % NIDXC

                @pl.when(chunk + 4 < NCH)
                def _():
                    idxf(chunk + 4, j4)
            return 0

        lax.fori_loop(0, NCH // NIDXC, step, 0)
        for chunk in range(NCH - NRC, NCH):
            b = chunk % NRC
            pltpu.make_async_copy(vals[b], c_sh.at[fidx[b]], ssem[b]).wait()
        plsc.subcore_barrier()
        for z in range(_CPT // _ZC):
            off = cbase + z * _ZC
            pltpu.sync_copy(c_sh.at[pl.ds(off, _ZC)], stage_v)
            pltpu.sync_copy(stage_v, out_hbm.at[pl.ds(off, _ZC)])

    @pl.when(c == 0)
    def _():
        run(invs_hbm, 0, 1, cf_out)

    @pl.when(c == 1)
    def _():
        run(invt_hbm, 1, 0, cb_out)


NROW = 5
NIDX = 10
LOOK = 3
RCH = RPT // KB

@functools.partial(
    pl.kernel,
    out_type=(_f32((NP, D)), _f32((NP, D))),
    mesh=_MESH,
    scratch_types=[
        *([pltpu.VMEM((2, KB), jnp.int32)] * NIDX),
        *([pltpu.VMEM((KB, D), jnp.float32)] * NROW),
        pltpu.VMEM_SHARED((NP, D), jnp.float32),
        *([pltpu.SemaphoreType.DMA] * (NROW + NROW + NIDX)),
    ],
    compiler_params=_SC_PARAMS,
)
def _sc_rows(xsf_hbm, xsb_hbm, pairs_hbm, zrows_hbm, accf_out, accb_out,
             *rest):
    ibuf = rest[:NIDX]
    rows = rest[NIDX:NIDX + NROW]
    acc_sh = rest[NIDX + NROW]
    gsem = rest[NIDX + NROW + 1:NIDX + NROW + 1 + NROW]
    ssem = rest[NIDX + NROW + 1 + NROW:NIDX + NROW + 1 + 2 * NROW]
    isem = rest[NIDX + NROW + 1 + 2 * NROW:]
    s = lax.axis_index("s")
    c = lax.axis_index("c")

    def run(table_hbm, gi, si, out_hbm):
        pltpu.sync_copy(zrows_hbm, rows[0])
        for z in range(RCH):
            pltpu.sync_copy(rows[0], acc_sh.at[pl.ds(s * RPT + z * KB, KB)])
        plsc.subcore_barrier()

        def idxf(chunk, j):
            return pltpu.async_copy(pairs_hbm.at[s].at[chunk], ibuf[j],
                                    isem[j])

        def gather(j, b):
            return pltpu.async_copy(table_hbm.at[ibuf[j].at[gi]], rows[b],
                                    gsem[b])

        def scatter(j, b):
            return pltpu.async_copy(rows[b], acc_sh.at[ibuf[j].at[si]],
                                    ssem[b], add=True)

        cps = [idxf(cc, cc) for cc in range(NROW)]
        for cc in range(LOOK):
            cps[cc].wait()
            gather(cc, cc)

        def step(k, _):
            for j in range(NIDX):
                chunk = k * NIDX + j
                b = j % NROW
                pltpu.make_async_copy(table_hbm.at[ibuf[j].at[gi]], rows[b],
                                      gsem[b]).wait()
                scatter(j, b)
                j2 = (j + LOOK) % NIDX
                b2 = (j + LOOK) % NROW

                @pl.when(chunk + LOOK < NCHB)
                def _():
                    @pl.when(chunk >= NROW - LOOK)
                    def _():
                        pltpu.make_async_copy(
                            rows[b2], acc_sh.at[ibuf[j2].at[si]],
                            ssem[b2]).wait()
                    pltpu.make_async_copy(pairs_hbm.at[s].at[chunk],
                                          ibuf[j2], isem[j2]).wait()
                    gather(j2, b2)

                j4 = (j + NROW) % NIDX

                @pl.when(chunk + NROW < NCHB)
                def _():
                    idxf(chunk + NROW, j4)
            return 0

        lax.fori_loop(0, NCHB // NIDX, step, 0)
        for chunk in range(NCHB - NROW, NCHB):
            j, b = chunk % NIDX, chunk % NROW
            pltpu.make_async_copy(rows[b], acc_sh.at[ibuf[j].at[si]],
                                  ssem[b]).wait()
        plsc.subcore_barrier()
        for z in range(RCH):
            off = s * RPT + z * KB
            pltpu.sync_copy(acc_sh.at[pl.ds(off, KB)], rows[0])
            pltpu.sync_copy(rows[0], out_hbm.at[pl.ds(off, KB)])

    @pl.when(c == 0)
    def _():
        run(xsf_hbm, 0, 1, accf_out)

    @pl.when(c == 1)
    def _():
        run(xsb_hbm, 1, 0, accb_out)


def _tc_prescale_body(x_ref, degs_ref, degt_ref, invs_ref, invt_ref,
                      xsf_ref, xsb_ref):
    invs = lax.rsqrt(jnp.maximum(degs_ref[...], 1.0))
    invt = lax.rsqrt(jnp.maximum(degt_ref[...], 1.0))
    invs_ref[...] = invs
    invt_ref[...] = invt
    x = x_ref[...]
    xsf_ref[...] = x * invs
    xsb_ref[...] = x * invt


def _tc_epilogue_body(x_ref, accf_ref, accb_ref, cf_ref, cb_ref, invs_ref,
                      invt_ref, xre_ref, xro_ref, wl_ref, wf_ref, wb_ref,
                      xr_ref, wr_ref, sl_ref, bias_ref, gamma_ref, beta_ref,
                      h_ref, xr_new_ref):
    f32 = jnp.float32
    relf = jnp.dot(cf_ref[...], xre_ref[...], preferred_element_type=f32)
    relb = jnp.dot(cb_ref[...], xro_ref[...], preferred_element_type=f32)
    fwd = jnp.dot(invt_ref[...] * (accf_ref[...] - relf), wf_ref[...],
                  preferred_element_type=f32)
    bwd = jnp.dot(invs_ref[...] * (accb_ref[...] - relb), wb_ref[...],
                  preferred_element_type=f32)
    loop = jnp.dot(x_ref[...] - sl_ref[...], wl_ref[...],
                   preferred_element_type=f32)
    h = (loop + fwd + bwd) * (1.0 / 3.0) + bias_ref[...]
    rows = lax.broadcasted_iota(jnp.int32, (NP, 1), 0)
    h = jnp.where(rows < N, h, 0.0)
    mean = jnp.sum(h, axis=0, keepdims=True) * (1.0 / N)
    var = jnp.sum(h * h, axis=0, keepdims=True) * (1.0 / N) - mean * mean
    inv_std = lax.rsqrt(var + 1e-5)
    h_ref[...] = (h - mean) * inv_std * gamma_ref[...] + beta_ref[...]
    xr_new_ref[...] = lax.dot_general(
        xr_ref[...], wr_ref[...], (((1,), (1,)), ((), ())),
        preferred_element_type=f32)


_tc_prescale = pl.pallas_call(
    _tc_prescale_body,
    out_shape=(_f32((NP, 1)), _f32((NP, 1)), _f32((NP, D)), _f32((NP, D))),
)

_tc_epilogue = pl.pallas_call(
    _tc_epilogue_body,
    out_shape=(_f32((NP, D)), _f32((200, D))),
)


def kernel(x_e, x_r, edge_index, edge_type, w_loop, w_fwd, w_bwd, w_rel,
           self_loop, bias, bn_gamma, bn_beta):
    pad = EPAD - E
    pidx = jnp.arange(pad, dtype=jnp.int32)
    src = jnp.concatenate([edge_index[0], N + pidx % (NP - N)])
    tgt = jnp.concatenate([edge_index[1], N + (pidx + 97) % (NP - N)])
    et = jnp.concatenate([edge_type, jnp.zeros((pad,), jnp.int32)])

    x_e_p = jnp.concatenate([x_e, jnp.zeros((NP - N, D), jnp.float32)])

    pairs = jnp.stack([src.reshape(NSUB, NCHB, KB),
                       tgt.reshape(NSUB, NCHB, KB)], axis=2)
    triple = jnp.stack([src.reshape(NSUB, NCH, K),
                        tgt.reshape(NSUB, NCH, K),
                        et.reshape(NSUB, NCH, K)], axis=2)

    deg_s, deg_t = _sc_degrees(pairs)

    invs, invt, xs_f, xs_b = _tc_prescale(
        x_e_p, deg_s.reshape(NP, 1), deg_t.reshape(NP, 1))

    cf_flat, cb_flat = _sc_coeff(triple, invs.reshape(NP), invt.reshape(NP))

    zrows = jnp.zeros((KB, D), jnp.float32)
    acc_f, acc_b = _sc_rows(xs_f, xs_b, pairs, zrows)

    xr_even = jnp.concatenate([x_r[0::2], jnp.zeros((CW - NREL, D))])
    xr_odd = jnp.concatenate([x_r[1::2], jnp.zeros((CW - NREL, D))])

    h_full, x_r_new = _tc_epilogue(
        x_e_p, acc_f, acc_b, cf_flat.reshape(NP, CW), cb_flat.reshape(NP, CW),
        invs, invt, xr_even, xr_odd, w_loop, w_fwd, w_bwd, x_r, w_rel,
        self_loop, bias.reshape(1, D), bn_gamma.reshape(1, D),
        bn_beta.reshape(1, D))

    return h_full[:N], x_r_new
